# baseline (speedup 1.0000x reference)
import functools

import jax
import jax.numpy as jnp
from jax import lax
from jax.experimental import pallas as pl
from jax.experimental.pallas import tpu as pltpu
from jax.experimental.pallas import tpu_sc as plsc

MAXLEN = 200
EMBED_DIM = 64
BATCH = 4096
NROWS = MAXLEN * EMBED_DIM
LANES = 16
NUM_CORES = 2
NUM_SUBCORES = 16
NUM_WORKERS = NUM_CORES * NUM_SUBCORES
ROWS_PER_WORKER = NROWS // NUM_WORKERS
RCHUNK = 4
NCHUNK = ROWS_PER_WORKER // RCHUNK
VSLICES = BATCH // LANES
NBUF = 3
N_OUTER = (NCHUNK + NBUF - 1) // NBUF

_mesh = plsc.VectorSubcoreMesh(core_axis_name="c", subcore_axis_name="s")

_VMEM_BUF = pltpu.VMEM((RCHUNK, BATCH), jnp.float32)


@functools.partial(
    pl.kernel,
    mesh=_mesh,
    out_type=jax.ShapeDtypeStruct((NROWS, BATCH), jnp.float32),
    scratch_types=(
        [pltpu.VMEM((ROWS_PER_WORKER + LANES,), jnp.float32)]
        + [_VMEM_BUF] * (2 * NBUF)
        + [pltpu.SemaphoreType.DMA] * (2 * NBUF)
    ),
)
def _sc_add(x_hbm, pos_hbm, out_hbm, pos_v, *scratch):
    ins = scratch[:NBUF]
    ous = scratch[NBUF:2 * NBUF]
    sis = scratch[2 * NBUF:3 * NBUF]
    sos = scratch[3 * NBUF:4 * NBUF]
    wid = lax.axis_index("s") * NUM_CORES + lax.axis_index("c")
    base = wid * ROWS_PER_WORKER
    pltpu.sync_copy(pos_hbm.at[pl.ds(base, ROWS_PER_WORKER)],
                    pos_v.at[pl.ds(0, ROWS_PER_WORKER)])

    for b in range(NBUF):
        pltpu.async_copy(x_hbm.at[pl.ds(base + b * RCHUNK, RCHUNK)],
                         ins[b], sis[b])

    def outer(i, carry):
        c0 = i * NBUF
        for b in range(NBUF):
            c = c0 + b

            @pl.when(c < NCHUNK)
            def _():
                row0 = base + c * RCHUNK
                in_b = ins[b]
                ou_b = ous[b]

                pltpu.make_async_copy(
                    x_hbm.at[pl.ds(row0, RCHUNK)], in_b, sis[b]
                ).wait()

                @pl.when(c >= NBUF)
                def _():
                    pltpu.make_async_copy(
                        ou_b, out_hbm.at[pl.ds(row0, RCHUNK)], sos[b]
                    ).wait()

                pvec = pos_v[pl.ds(c * RCHUNK, LANES)]
                for r in range(RCHUNK):
                    pv = jnp.full((LANES,), pvec[r], jnp.float32)

                    @plsc.parallel_loop(0, VSLICES, 1, unroll=8)
                    def _(j):
                        ou_b[r, pl.ds(j * LANES, LANES)] = (
                            in_b[r, pl.ds(j * LANES, LANES)] + pv
                        )

                @pl.when(c < NCHUNK - NBUF)
                def _():
                    pltpu.async_copy(
                        x_hbm.at[pl.ds(row0 + NBUF * RCHUNK, RCHUNK)],
                        in_b, sis[b],
                    )

                pltpu.async_copy(ou_b, out_hbm.at[pl.ds(row0, RCHUNK)],
                                 sos[b])
        return carry

    lax.fori_loop(0, N_OUTER, outer, 0)

    for k in range(NCHUNK - NBUF, NCHUNK):
        row0 = base + k * RCHUNK
        pltpu.make_async_copy(ous[k % NBUF],
                              out_hbm.at[pl.ds(row0, RCHUNK)],
                              sos[k % NBUF]).wait()


def kernel(x, pos_table):
    x2 = jnp.transpose(x, (1, 2, 0)).reshape(NROWS, BATCH)
    pos_flat = pos_table.reshape(NROWS)
    out2 = _sc_add(x2, pos_flat)
    return jnp.transpose(out2.reshape(MAXLEN, EMBED_DIM, BATCH), (2, 0, 1))
